# Initial kernel scaffold; baseline (speedup 1.0000x reference)
import functools

import jax
import jax.numpy as jnp
from jax import lax
from jax.experimental import pallas as pl
from jax.experimental.pallas import tpu as pltpu
from jax.experimental.pallas import tpu_sc as plsc

EMB = 50
NC, NS = 2, 16
NW = NC * NS
IDX_TILE = 128
TILES_PER_CHUNK = 8
CHUNK = IDX_TILE * TILES_PER_CHUNK


@functools.cache
def _make(B):
    assert B % (NW * CHUNK) == 0
    b_per_w = B // NW
    n_chunks = b_per_w // CHUNK
    rows_per_w = b_per_w // IDX_TILE
    mesh = plsc.VectorSubcoreMesh(core_axis_name="c", subcore_axis_name="s")

    @functools.partial(
        pl.kernel,
        mesh=mesh,
        out_type=jax.ShapeDtypeStruct((B, EMB), jnp.float32),
        scratch_types=[
            pltpu.VMEM((TILES_PER_CHUNK, IDX_TILE), jnp.int32),
            pltpu.VMEM((CHUNK, EMB), jnp.float32),
            pltpu.SemaphoreType.DMA,
        ],
    )
    def k(batch_hbm, table_hbm, out_hbm, idx_v, rows_v, sem):
        wid = lax.axis_index("s") * NC + lax.axis_index("c")

        def step(i, carry):
            base = wid * b_per_w + i * CHUNK
            rowbase = wid * rows_per_w + i * TILES_PER_CHUNK
            pltpu.sync_copy(batch_hbm.at[pl.ds(rowbase, TILES_PER_CHUNK)], idx_v)
            cps = []
            for j in range(TILES_PER_CHUNK):
                cp = pltpu.make_async_copy(
                    table_hbm.at[idx_v.at[j]],
                    rows_v.at[pl.ds(j * IDX_TILE, IDX_TILE)],
                    sem,
                )
                cp.start()
                cps.append(cp)
            for cp in cps:
                cp.wait()
            pltpu.sync_copy(rows_v, out_hbm.at[pl.ds(base, CHUNK)])
            return carry

        lax.fori_loop(0, n_chunks, step, 0)

    return k


def kernel(batch, table):
    R, C = batch.shape
    B = R * C
    batch2 = batch.reshape(B // IDX_TILE, IDX_TILE).astype(jnp.int32)
    out = _make(B)(batch2, table)
    return out.reshape(R, C, EMB)


# speedup vs baseline: 4.2828x; 4.2828x over previous
import functools

import jax
import jax.numpy as jnp
from jax import lax
from jax.experimental import pallas as pl
from jax.experimental.pallas import tpu as pltpu
from jax.experimental.pallas import tpu_sc as plsc

EMB = 50
WPAD = 64
NC, NS = 2, 16
NW = NC * NS
IDX_TILE = 128
TILES_PER_CHUNK = 8
CHUNK = IDX_TILE * TILES_PER_CHUNK


@functools.cache
def _make(B):
    assert B % (NW * CHUNK) == 0
    b_per_w = B // NW
    n_chunks = b_per_w // CHUNK
    mesh = plsc.VectorSubcoreMesh(core_axis_name="c", subcore_axis_name="s")

    @functools.partial(
        pl.kernel,
        mesh=mesh,
        out_type=jax.ShapeDtypeStruct((B, EMB), jnp.float32),
        compiler_params=pltpu.CompilerParams(use_tc_tiling_on_sc=False),
        scratch_types=[
            pltpu.VMEM((TILES_PER_CHUNK, IDX_TILE), jnp.int32),
            pltpu.VMEM((CHUNK, WPAD), jnp.float32),
            pltpu.VMEM((CHUNK, EMB), jnp.float32),
            pltpu.VMEM_SHARED((128, WPAD), jnp.float32),
            pltpu.SemaphoreType.DMA,
        ],
    )
    def k(batch_hbm, table_hbm, out_hbm, idx_v, rows_v, dense_v, table_sh, sem):
        s = lax.axis_index("s")
        wid = s * NC + lax.axis_index("c")

        @pl.when(s == 0)
        def _():
            pltpu.sync_copy(table_hbm, table_sh)

        plsc.subcore_barrier()

        def step(i, carry):
            base = wid * b_per_w + i * CHUNK
            rowbase = base // IDX_TILE
            pltpu.sync_copy(batch_hbm.at[pl.ds(rowbase, TILES_PER_CHUNK)], idx_v)
            cps = []
            for j in range(TILES_PER_CHUNK):
                cp = pltpu.make_async_copy(
                    table_sh.at[idx_v.at[j]],
                    rows_v.at[pl.ds(j * IDX_TILE, IDX_TILE)],
                    sem,
                )
                cp.start()
                cps.append(cp)
            for cp in cps:
                cp.wait()

            @plsc.parallel_loop(0, CHUNK, unroll=4)
            def _row(r):
                for off in (0, 16, 32, 34):
                    dense_v[r, pl.ds(off, 16)] = rows_v[r, pl.ds(off, 16)]

            pltpu.sync_copy(dense_v, out_hbm.at[pl.ds(base, CHUNK)])
            return carry

        lax.fori_loop(0, n_chunks, step, 0)

    return k


def kernel(batch, table):
    R, C = batch.shape
    B = R * C
    flat = batch.reshape(B // IDX_TILE, IDX_TILE).astype(jnp.int32)
    tpad = jnp.zeros((table.shape[0], WPAD), jnp.float32).at[:, :EMB].set(table)
    out = _make(B)(flat, tpad)
    return out.reshape(R, C, EMB)
